# Initial kernel scaffold; baseline (speedup 1.0000x reference)
import jax
import jax.numpy as jnp
from jax.experimental import pallas as pl


def kernel(x, edge_index, emb, W1, b1, W2, b2, gamma, beta):
    raise NotImplementedError("write your pallas kernel here")


# speedup vs baseline: 5.0408x; 5.0408x over previous
import functools

import jax
import jax.numpy as jnp
from jax import lax
from jax.experimental import pallas as pl
from jax.experimental.pallas import tpu as pltpu
from jax.experimental.pallas import tpu_sc as plsc

N = 10000
E = 320000
D = 128
L = 5
NW = 32
NP = 10240
C = 128
EPT = 10112
EP = EPT * NW
RPT = NP // 16
C0 = 80

_mesh = plsc.VectorSubcoreMesh(core_axis_name="c", subcore_axis_name="s")


@functools.partial(
    pl.kernel,
    mesh=_mesh,
    out_type=jax.ShapeDtypeStruct((NP, D), jnp.float32),
    scratch_types=[
        pltpu.VMEM((C0,), jnp.int32),
        pltpu.VMEM((C0, D), jnp.float32),
        pltpu.SemaphoreType.DMA,
    ],
)
def _emb_gather(x_hbm, emb_hbm, out_hbm, idx_v, rows_v, sem):
    c = lax.axis_index("c")
    s = lax.axis_index("s")
    wid = s * 2 + c
    base = wid * (NP // NW)
    for k in range(NP // NW // C0):
        off = base + k * C0
        pltpu.sync_copy(x_hbm.at[pl.ds(off, C0)], idx_v)
        pltpu.async_copy(emb_hbm.at[idx_v], rows_v, sem).wait()
        pltpu.sync_copy(rows_v, out_hbm.at[pl.ds(off, C0)])


@functools.partial(
    pl.kernel,
    mesh=_mesh,
    out_type=jax.ShapeDtypeStruct((2 * NP, D), jnp.float32),
    scratch_types=[
        pltpu.VMEM((C,), jnp.int32),
        pltpu.VMEM((C,), jnp.int32),
        pltpu.VMEM((C, D), jnp.float32),
        pltpu.VMEM_SHARED((NP, D), jnp.float32),
        pltpu.SemaphoreType.DMA,
    ],
)
def _agg(h_hbm, zeros_hbm, src_hbm, dst_hbm, out_hbm, src_v, dst_v, rows_v,
         acc_sh, sem):
    c = lax.axis_index("c")
    s = lax.axis_index("s")
    rbase = s * RPT

    @pl.when(c == 0)
    def _():
        pltpu.sync_copy(h_hbm.at[pl.ds(rbase, RPT)], acc_sh.at[pl.ds(rbase, RPT)])

    @pl.when(c == 1)
    def _():
        pltpu.sync_copy(zeros_hbm.at[pl.ds(rbase, RPT)], acc_sh.at[pl.ds(rbase, RPT)])

    plsc.subcore_barrier()

    wid = s * 2 + c
    ebase = wid * EPT

    def body(k, carry):
        off = ebase + k * C
        pltpu.sync_copy(src_hbm.at[pl.ds(off, C)], src_v)
        pltpu.sync_copy(dst_hbm.at[pl.ds(off, C)], dst_v)
        pltpu.async_copy(h_hbm.at[src_v], rows_v, sem).wait()
        pltpu.sync_copy(rows_v, acc_sh.at[dst_v], add=True)
        return carry

    lax.fori_loop(0, EPT // C, body, 0)
    plsc.subcore_barrier()
    pltpu.sync_copy(acc_sh.at[pl.ds(rbase, RPT)],
                    out_hbm.at[pl.ds(c * NP + rbase, RPT)])


R = 640


def _mlp_body(last, p_ref, w1_ref, b1_ref, w2_ref, b2_ref, o_ref):
    agg = p_ref[0] + p_ref[1]
    hid = jnp.dot(agg, w1_ref[...], preferred_element_type=jnp.float32)
    hid = jnp.maximum(hid + b1_ref[...], 0.0)
    out = jnp.dot(hid, w2_ref[...], preferred_element_type=jnp.float32)
    out = out + b2_ref[...]
    if not last:
        out = jnp.maximum(out, 0.0)
    rows = R * pl.program_id(0) + lax.broadcasted_iota(jnp.int32, (R, 1), 0)
    o_ref[...] = jnp.where(rows < N, out, 0.0)


def _make_mlp(last):
    return pl.pallas_call(
        functools.partial(_mlp_body, last),
        grid=(NP // R,),
        in_specs=[
            pl.BlockSpec((2, R, D), lambda i: (0, i, 0)),
            pl.BlockSpec((D, 2 * D), lambda i: (0, 0)),
            pl.BlockSpec((1, 2 * D), lambda i: (0, 0)),
            pl.BlockSpec((2 * D, D), lambda i: (0, 0)),
            pl.BlockSpec((1, D), lambda i: (0, 0)),
        ],
        out_specs=pl.BlockSpec((R, D), lambda i: (i, 0)),
        out_shape=jax.ShapeDtypeStruct((NP, D), jnp.float32),
    )


_mlp_mid = _make_mlp(False)
_mlp_last = _make_mlp(True)


def kernel(x, edge_index, emb, W1, b1, W2, b2, gamma, beta):
    inv = jnp.float32(1.0) / jnp.sqrt(jnp.float32(1.0 + 1e-5))
    scale = gamma * inv
    W2p = W2 * scale[:, None, :]
    b2p = b2 * scale + beta

    x_pad = jnp.concatenate([x, jnp.zeros((NP - N,), jnp.int32)])
    src = jnp.concatenate([edge_index[0], jnp.zeros((EP - E,), jnp.int32)])
    dst = jnp.concatenate([edge_index[1], jnp.full((EP - E,), N, jnp.int32)])
    zeros = jnp.zeros((NP, D), jnp.float32)

    h = _emb_gather(x_pad, emb)
    for l in range(L):
        parts = _agg(h, zeros, src, dst).reshape(2, NP, D)
        mlp = _mlp_last if l == L - 1 else _mlp_mid
        h = mlp(parts, W1[l], b1[l].reshape(1, -1), W2p[l], b2p[l].reshape(1, -1))
    return h[:N]
